# Initial kernel scaffold; baseline (speedup 1.0000x reference)
import jax
import jax.numpy as jnp
from jax.experimental import pallas as pl


def kernel(Jraw, J_transf):
    raise NotImplementedError("write your pallas kernel here")


# speedup vs baseline: 21.7439x; 21.7439x over previous
import functools

import jax
import jax.numpy as jnp
from jax import lax
from jax.experimental import pallas as pl
from jax.experimental.pallas import tpu as pltpu
from jax.experimental.pallas import tpu_sc as plsc

N = 1024
N_OFF = N * (N - 1) // 2
OFF2 = N * (N + 1) // 2
NC, NS = 2, 16
NW = NC * NS
ROWS_PER_W = N // NW
CHUNK = 128
NCHUNK = N // CHUNK


def _build(jraw2, jt):
    mesh = plsc.VectorSubcoreMesh(core_axis_name="c", subcore_axis_name="s")

    @functools.partial(
        pl.kernel,
        mesh=mesh,
        compiler_params=pltpu.CompilerParams(
            needs_layout_passes=False, use_tc_tiling_on_sc=False),
        out_type=jax.ShapeDtypeStruct((N * N, 4), jnp.float32),
        scratch_types=[
            pltpu.VMEM((N,), jnp.int32),
            pltpu.VMEM((NCHUNK, CHUNK), jnp.int32),
            pltpu.VMEM((N, 4), jnp.float32),
            pltpu.SemaphoreType.DMA,
        ],
    )
    def k(jraw_hbm, jt_hbm, out_hbm, idx_v, idx2_v, stage_v, sem):
        wid = lax.axis_index("s") * NC + lax.axis_index("c")
        lane = lax.iota(jnp.int32, 16)
        sub = lane & 3
        rbase = lane >> 2
        cflip = jnp.where(sub == 1, 2, jnp.where(sub == 2, 1, sub))

        def row_body(i, carry):
            j = i * NW + wid

            pltpu.sync_copy(jt_hbm.at[pl.ds(j * N, N)], idx_v)

            for c in range(NCHUNK):
                def xform(m, cc, c=c):
                    v = idx_v[pl.ds(c * CHUNK + m * 16, 16)]
                    idx2_v[c, pl.ds(m * 16, 16)] = jnp.where(
                        v >= OFF2, v - OFF2, jnp.where(v >= N_OFF, 0, v))
                    return cc
                lax.fori_loop(0, CHUNK // 16, xform, 0)

            cps = [
                pltpu.async_copy(
                    jraw_hbm.at[idx2_v.at[c]],
                    stage_v.at[pl.ds(c * CHUNK, CHUNK)],
                    sem,
                )
                for c in range(NCHUNK)
            ]
            for cp in cps:
                cp.wait()

            def flip(kk, c):
                rv = kk * 4 + rbase
                v = plsc.load_gather(stage_v, [rv, cflip])
                plsc.store_scatter(stage_v, [rv, sub], v)
                return c
            lax.fori_loop(0, j >> 2, flip, 0)
            rv = (j >> 2) * 4 + rbase
            v = plsc.load_gather(stage_v, [rv, cflip])
            v = jnp.where(rv == j, 0.0, v)
            plsc.store_scatter(stage_v, [rv, sub], v, mask=rv <= j)

            pltpu.sync_copy(stage_v, out_hbm.at[pl.ds(j * N, N)])
            return carry

        lax.fori_loop(0, ROWS_PER_W, row_body, 0)

    return k(jraw2, jt)


def kernel(Jraw, J_transf):
    jraw2 = Jraw.reshape(N_OFF, 4)
    out = _build(jraw2, J_transf.astype(jnp.int32))
    return out.reshape(N, N, 2, 2)
